# baseline (speedup 1.0000x reference)
import functools

import jax
import jax.numpy as jnp
from jax import lax
from jax.experimental import pallas as pl
from jax.experimental.pallas import tpu as pltpu
from jax.experimental.pallas import tpu_sc as plsc

B = 1; T = 2048; D = 768; H = 12; HD = 64; E = 8; K = 2; V = 8192
HID = 1536; BLK = 2048; EPS = 1e-5
CAP = (B * T * K) // E
ECAP = E * CAP
HHD = H * HD
BT = 256
HC = 512
VB = 512


def _rms_rows(x, w):
    return x * jax.lax.rsqrt(jnp.mean(x * x, axis=-1, keepdims=True) + EPS) * w


def _attn_prefix(x, Wq, bq, Wk, bk, Wv, bv, Wo, bo, ln1):
    xn = _rms_rows(x, ln1)
    q = jnp.einsum('btc,hdc->bhtd', xn, Wq) + bq[None, :, None, :]
    k = jnp.einsum('btc,hdc->bhtd', xn, Wk) + bk[None, :, None, :]
    v = jnp.einsum('btc,hdc->bhtd', xn, Wv) + bv[None, :, None, :]
    inv_freq = 1.0 / (10000.0 ** (jnp.arange(0, HD, 2, dtype=jnp.float32) / HD))
    pos = jnp.arange(BLK, dtype=jnp.float32)
    sinus = pos[:, None] * inv_freq[None, :]
    cos = jnp.cos(sinus)[:T][None, None, :, :]
    sin = jnp.sin(sinus)[:T][None, None, :, :]
    q1, q2 = q[..., 0::2], q[..., 1::2]
    k1, k2 = k[..., 0::2], k[..., 1::2]
    q = jnp.concatenate([q1 * cos - q2 * sin, q1 * sin + q2 * cos], axis=-1)
    k = jnp.concatenate([k1 * cos - k2 * sin, k1 * sin + k2 * cos], axis=-1)
    scores = jnp.einsum('bhtd,bhsd->bhts', q, k) / (HD ** 0.5)
    mask = jnp.tril(jnp.ones((T, T), dtype=bool))
    scores = jnp.where(mask[None, None, :, :], scores, -jnp.inf)
    att = jax.nn.softmax(scores, axis=-1)
    out = jnp.einsum('bhts,bhsd->bhtd', att, v)
    out = out.transpose(0, 2, 1, 3).reshape(B, T, H * HD)
    return x + (out @ Wo.T + bo)


def _top2(noisy, n, lane):
    v1 = jnp.max(noisy, axis=-1, keepdims=True)
    i1 = jnp.min(jnp.where(noisy == v1, lane, E), axis=-1, keepdims=True)
    masked = jnp.where(lane == i1, -jnp.inf, noisy)
    v2 = jnp.max(masked, axis=-1, keepdims=True)
    i2 = jnp.min(jnp.where(masked == v2, lane, E), axis=-1, keepdims=True)
    return i1, i2, v1, v2


def _sel_body(noisyf_ref, noisy_ref, ri_ref, rg_ref, s1_ref, s2_ref):
    rb = pl.program_id(0)
    lane_f = jax.lax.broadcasted_iota(jnp.int32, (T, E), 1)
    i1f, i2f, _, _ = _top2(noisyf_ref[...], T, lane_f)
    sel = ((lane_f == i1f) | (lane_f == i2f)).astype(jnp.float32)

    rows = rb * BT + jax.lax.broadcasted_iota(jnp.int32, (BT, T), 0)
    cols = jax.lax.broadcasted_iota(jnp.int32, (BT, T), 1)
    tl = (cols < rows).astype(jnp.float32)
    pos = jnp.dot(tl, sel, preferred_element_type=jnp.float32).astype(jnp.int32)

    lane = jax.lax.broadcasted_iota(jnp.int32, (BT, E), 1)
    i1, i2, v1, v2 = _top2(noisy_ref[...], BT, lane)
    z = jnp.exp(v2 - v1)
    g1 = 1.0 / (1.0 + z)
    g2 = z / (1.0 + z)
    ri_ref[...] = jnp.where(lane == 0, i1, jnp.where(lane == 1, i2, 0))
    rg_ref[...] = jnp.where(lane == 0, g1, jnp.where(lane == 1, g2, 0.0))

    p1 = jnp.sum(jnp.where(lane == i1, pos, 0), axis=-1, keepdims=True)
    p2 = jnp.sum(jnp.where(lane == i2, pos, 0), axis=-1, keepdims=True)
    s1_ref[...] = jnp.where(p1 < CAP, i1 * CAP + p1, ECAP)
    s2_ref[...] = jnp.where(p2 < CAP, i2 * CAP + p2, ECAP)


def _sel_call(noisy):
    row = lambda i: (i, 0)
    full = lambda i: (0, 0)
    return pl.pallas_call(
        _sel_body,
        grid=(T // BT,),
        in_specs=[pl.BlockSpec((T, E), full), pl.BlockSpec((BT, E), row)],
        out_specs=[pl.BlockSpec((BT, E), row), pl.BlockSpec((BT, E), row),
                   pl.BlockSpec((BT, 1), row), pl.BlockSpec((BT, 1), row)],
        out_shape=[jax.ShapeDtypeStruct((T, E), jnp.int32),
                   jax.ShapeDtypeStruct((T, E), jnp.float32),
                   jax.ShapeDtypeStruct((T, 1), jnp.int32),
                   jax.ShapeDtypeStruct((T, 1), jnp.int32)],
    )(noisy, noisy)


def _ffn_body(x_ref, w1_ref, w3_ref, w2_ref, o_ref):
    x = x_ref[...]
    dn = (((1,), (1,)), ((), ()))
    a = jax.lax.dot_general(x, w1_ref[0], dn, preferred_element_type=jnp.float32)
    b = jax.lax.dot_general(x, w3_ref[0], dn, preferred_element_type=jnp.float32)
    h = (a * jax.nn.sigmoid(a)) * b
    o_ref[...] = jax.lax.dot_general(h, w2_ref[0], dn,
                                     preferred_element_type=jnp.float32)


def _ffn_call(xdisp, w1, w3, w2):
    return pl.pallas_call(
        _ffn_body,
        grid=(E,),
        in_specs=[
            pl.BlockSpec((CAP, D), lambda e: (e, 0)),
            pl.BlockSpec((1, HID, D), lambda e: (e, 0, 0)),
            pl.BlockSpec((1, HID, D), lambda e: (e, 0, 0)),
            pl.BlockSpec((1, D, HID), lambda e: (e, 0, 0)),
        ],
        out_specs=pl.BlockSpec((CAP, D), lambda e: (e, 0)),
        out_shape=jax.ShapeDtypeStruct((ECAP, D), jnp.float32),
    )(xdisp, w1, w3, w2)


def _lm_body(x1_ref, y1_ref, y2_ref, rg_ref, s1_ref, s2_ref, lnf_ref, w_ref,
             o_ref, xf_ref):
    vb = pl.program_id(0)

    @pl.when(vb == 0)
    def _():
        k1 = s1_ref[...] < ECAP
        k2 = s2_ref[...] < ECAP
        g1 = rg_ref[..., 0:1]
        g2 = rg_ref[..., 1:2]
        moe = (jnp.where(k1, y1_ref[...], 0.0) * jnp.where(k1, g1, 0.0)
               + jnp.where(k2, y2_ref[...], 0.0) * jnp.where(k2, g2, 0.0))
        x2 = x1_ref[...] + moe
        xf_ref[...] = _rms_rows(x2, lnf_ref[...])

    o_ref[...] = jnp.dot(xf_ref[...], w_ref[...],
                         preferred_element_type=jnp.float32)


def _lm_call(x1, y1, y2, rg, s1, s2, lnf, wlmT):
    full = lambda vb: (0, 0)
    return pl.pallas_call(
        _lm_body,
        grid=(V // VB,),
        in_specs=[
            pl.BlockSpec((T, D), full),
            pl.BlockSpec((T, D), full),
            pl.BlockSpec((T, D), full),
            pl.BlockSpec((T, E), full),
            pl.BlockSpec((T, 1), full),
            pl.BlockSpec((T, 1), full),
            pl.BlockSpec((1, D), full),
            pl.BlockSpec((D, VB), lambda vb: (0, vb)),
        ],
        out_specs=pl.BlockSpec((T, VB), lambda vb: (0, vb)),
        out_shape=jax.ShapeDtypeStruct((T, V), jnp.float32),
        scratch_shapes=[pltpu.VMEM((T, D), jnp.float32)],
    )(x1, y1, y2, rg, s1, s2, lnf, wlmT)


NC = 2
NS = 16
NW = NC * NS
TPW = T // NW

_SC_MESH = dict(core_axis_name="c", subcore_axis_name="s")


def _embed_gather(tok_emb, idxf):
    mesh = plsc.VectorSubcoreMesh(**_SC_MESH)

    @functools.partial(
        pl.kernel, mesh=mesh,
        out_type=jax.ShapeDtypeStruct((T, D), jnp.float32),
        scratch_types=[
            pltpu.VMEM((TPW,), jnp.int32),
            pltpu.VMEM((TPW, D), jnp.float32),
            pltpu.SemaphoreType.DMA,
        ],
    )
    def k(table_hbm, idx_hbm, out_hbm, idx_v, rows_v, sem):
        wid = lax.axis_index("s") * NC + lax.axis_index("c")
        base = wid * TPW
        pltpu.sync_copy(idx_hbm.at[pl.ds(base, TPW)], idx_v)
        pltpu.async_copy(table_hbm.at[idx_v], rows_v, sem).wait()
        pltpu.sync_copy(rows_v, out_hbm.at[pl.ds(base, TPW)])

    return k(tok_emb, idxf)


def _comb_gather(eo, slot1, slot2):
    mesh = plsc.VectorSubcoreMesh(**_SC_MESH)
    oy = jax.ShapeDtypeStruct((T, D), jnp.float32)

    @functools.partial(
        pl.kernel, mesh=mesh,
        out_type=[oy, oy],
        scratch_types=[
            pltpu.VMEM((TPW,), jnp.int32),
            pltpu.VMEM((TPW,), jnp.int32),
            pltpu.VMEM((TPW, D), jnp.float32),
            pltpu.VMEM((TPW, D), jnp.float32),
            pltpu.SemaphoreType.DMA,
            pltpu.SemaphoreType.DMA,
        ],
    )
    def k(eo_hbm, s1_hbm, s2_hbm, y1_hbm, y2_hbm, i1_v, i2_v, r1_v, r2_v,
          sem1, sem2):
        wid = lax.axis_index("s") * NC + lax.axis_index("c")
        base = wid * TPW
        pltpu.sync_copy(s1_hbm.at[pl.ds(base, TPW)], i1_v)
        pltpu.sync_copy(s2_hbm.at[pl.ds(base, TPW)], i2_v)
        for i in range(TPW // 16):
            sl = pl.ds(i * 16, 16)
            i1_v[sl] = jnp.minimum(i1_v[sl], ECAP - 1)
            i2_v[sl] = jnp.minimum(i2_v[sl], ECAP - 1)
        c1 = pltpu.async_copy(eo_hbm.at[i1_v], r1_v, sem1)
        c2 = pltpu.async_copy(eo_hbm.at[i2_v], r2_v, sem2)
        c1.wait()
        c2.wait()
        pltpu.sync_copy(r1_v, y1_hbm.at[pl.ds(base, TPW)])
        pltpu.sync_copy(r2_v, y2_hbm.at[pl.ds(base, TPW)])

    return k(eo, slot1, slot2)


DISP_ROWS = ECAP + NW


def _disp_scatter(slot1, slot2, x1n):
    mesh = plsc.VectorSubcoreMesh(**_SC_MESH)

    @functools.partial(
        pl.kernel, mesh=mesh,
        out_type=jax.ShapeDtypeStruct((DISP_ROWS, D), jnp.float32),
        scratch_types=[
            pltpu.VMEM((TPW,), jnp.int32),
            pltpu.VMEM((TPW,), jnp.int32),
            pltpu.VMEM((TPW, D), jnp.float32),
            pltpu.SemaphoreType.DMA,
        ],
    )
    def k(s1_hbm, s2_hbm, x_hbm, disp_hbm, d1_v, d2_v, rows_v, sem):
        cid = lax.axis_index("c")
        sid = lax.axis_index("s")
        wid = sid * NC + cid
        base = wid * TPW
        trash = ECAP + wid
        pltpu.sync_copy(s1_hbm.at[pl.ds(base, TPW)], d1_v)
        pltpu.sync_copy(s2_hbm.at[pl.ds(base, TPW)], d2_v)
        for j in range(TPW // 16):
            sl = pl.ds(j * 16, 16)
            d1_v[sl] = jnp.where(d1_v[sl] >= ECAP, trash, d1_v[sl])
            d2_v[sl] = jnp.where(d2_v[sl] >= ECAP, trash, d2_v[sl])
        pltpu.sync_copy(x_hbm.at[pl.ds(base, TPW)], rows_v)
        pltpu.async_copy(rows_v, disp_hbm.at[d1_v], sem).wait()
        pltpu.async_copy(rows_v, disp_hbm.at[d2_v], sem).wait()

    return k(slot1, slot2, x1n)


def kernel(idx, tok_emb, Wq, bq, Wk, bk, Wv, bv, Wo, bo, ln1, ln2, Wr, br, Wn, bn, w1, w3, w2, lnf, Wlm):
    x = _embed_gather(tok_emb, idx.reshape(T)).reshape(B, T, D)
    x1 = _attn_prefix(x, Wq, bq, Wk, bk, Wv, bv, Wo, bo, ln1)

    x1n = _rms_rows(x1, ln2)
    fx = x1n.reshape(T, D)
    logits = fx @ Wr.T + br
    nlog = fx @ Wn.T + bn
    eps = jax.random.normal(jax.random.key(42), (B, T, E), dtype=logits.dtype)
    noisy = logits + eps.reshape(T, E) * jax.nn.softplus(nlog)

    ri, rg, slot1, slot2 = _sel_call(noisy)
    s1f = slot1.reshape(T)
    s2f = slot2.reshape(T)
    xdisp = _disp_scatter(s1f, s2f, fx)

    eo = _ffn_call(xdisp, w1, w3, w2)

    y1, y2 = _comb_gather(eo, s1f, s2f)
    out = _lm_call(x1.reshape(T, D), y1, y2, rg, slot1, slot2,
                   lnf.reshape(1, D), Wlm.T)
    return out.reshape(B, T, V)


# speedup vs baseline: 1.0014x; 1.0014x over previous
import functools

import jax
import jax.numpy as jnp
from jax import lax
from jax.experimental import pallas as pl
from jax.experimental.pallas import tpu as pltpu
from jax.experimental.pallas import tpu_sc as plsc

B = 1; T = 2048; D = 768; H = 12; HD = 64; E = 8; K = 2; V = 8192
HID = 1536; BLK = 2048; EPS = 1e-5
CAP = (B * T * K) // E
ECAP = E * CAP
HHD = H * HD
BT = 256
HC = 512
VB = 1024


def _rms_rows(x, w):
    return x * jax.lax.rsqrt(jnp.mean(x * x, axis=-1, keepdims=True) + EPS) * w


def _attn_prefix(x, Wq, bq, Wk, bk, Wv, bv, Wo, bo, ln1):
    xn = _rms_rows(x, ln1)
    q = jnp.einsum('btc,hdc->bhtd', xn, Wq) + bq[None, :, None, :]
    k = jnp.einsum('btc,hdc->bhtd', xn, Wk) + bk[None, :, None, :]
    v = jnp.einsum('btc,hdc->bhtd', xn, Wv) + bv[None, :, None, :]
    inv_freq = 1.0 / (10000.0 ** (jnp.arange(0, HD, 2, dtype=jnp.float32) / HD))
    pos = jnp.arange(BLK, dtype=jnp.float32)
    sinus = pos[:, None] * inv_freq[None, :]
    cos = jnp.cos(sinus)[:T][None, None, :, :]
    sin = jnp.sin(sinus)[:T][None, None, :, :]
    q1, q2 = q[..., 0::2], q[..., 1::2]
    k1, k2 = k[..., 0::2], k[..., 1::2]
    q = jnp.concatenate([q1 * cos - q2 * sin, q1 * sin + q2 * cos], axis=-1)
    k = jnp.concatenate([k1 * cos - k2 * sin, k1 * sin + k2 * cos], axis=-1)
    scores = jnp.einsum('bhtd,bhsd->bhts', q, k) / (HD ** 0.5)
    mask = jnp.tril(jnp.ones((T, T), dtype=bool))
    scores = jnp.where(mask[None, None, :, :], scores, -jnp.inf)
    att = jax.nn.softmax(scores, axis=-1)
    out = jnp.einsum('bhts,bhsd->bhtd', att, v)
    out = out.transpose(0, 2, 1, 3).reshape(B, T, H * HD)
    return x + (out @ Wo.T + bo)


def _top2(noisy, n, lane):
    v1 = jnp.max(noisy, axis=-1, keepdims=True)
    i1 = jnp.min(jnp.where(noisy == v1, lane, E), axis=-1, keepdims=True)
    masked = jnp.where(lane == i1, -jnp.inf, noisy)
    v2 = jnp.max(masked, axis=-1, keepdims=True)
    i2 = jnp.min(jnp.where(masked == v2, lane, E), axis=-1, keepdims=True)
    return i1, i2, v1, v2


def _sel_body(noisyf_ref, noisy_ref, ri_ref, rg_ref, s1_ref, s2_ref):
    rb = pl.program_id(0)
    lane_f = jax.lax.broadcasted_iota(jnp.int32, (T, E), 1)
    i1f, i2f, _, _ = _top2(noisyf_ref[...], T, lane_f)
    sel = ((lane_f == i1f) | (lane_f == i2f)).astype(jnp.float32)

    rows = rb * BT + jax.lax.broadcasted_iota(jnp.int32, (BT, T), 0)
    cols = jax.lax.broadcasted_iota(jnp.int32, (BT, T), 1)
    tl = (cols < rows).astype(jnp.float32)
    pos = jnp.dot(tl, sel, preferred_element_type=jnp.float32).astype(jnp.int32)

    lane = jax.lax.broadcasted_iota(jnp.int32, (BT, E), 1)
    i1, i2, v1, v2 = _top2(noisy_ref[...], BT, lane)
    z = jnp.exp(v2 - v1)
    g1 = 1.0 / (1.0 + z)
    g2 = z / (1.0 + z)
    ri_ref[...] = jnp.where(lane == 0, i1, jnp.where(lane == 1, i2, 0))
    rg_ref[...] = jnp.where(lane == 0, g1, jnp.where(lane == 1, g2, 0.0))

    p1 = jnp.sum(jnp.where(lane == i1, pos, 0), axis=-1, keepdims=True)
    p2 = jnp.sum(jnp.where(lane == i2, pos, 0), axis=-1, keepdims=True)
    s1_ref[...] = jnp.where(p1 < CAP, i1 * CAP + p1, ECAP)
    s2_ref[...] = jnp.where(p2 < CAP, i2 * CAP + p2, ECAP)


def _sel_call(noisy):
    row = lambda i: (i, 0)
    full = lambda i: (0, 0)
    return pl.pallas_call(
        _sel_body,
        grid=(T // BT,),
        in_specs=[pl.BlockSpec((T, E), full), pl.BlockSpec((BT, E), row)],
        out_specs=[pl.BlockSpec((BT, E), row), pl.BlockSpec((BT, E), row),
                   pl.BlockSpec((BT, 1), row), pl.BlockSpec((BT, 1), row)],
        out_shape=[jax.ShapeDtypeStruct((T, E), jnp.int32),
                   jax.ShapeDtypeStruct((T, E), jnp.float32),
                   jax.ShapeDtypeStruct((T, 1), jnp.int32),
                   jax.ShapeDtypeStruct((T, 1), jnp.int32)],
    )(noisy, noisy)


def _ffn_body(x_ref, w1_ref, w3_ref, w2_ref, o_ref):
    x = x_ref[...]
    dn = (((1,), (1,)), ((), ()))
    a = jax.lax.dot_general(x, w1_ref[0], dn, preferred_element_type=jnp.float32)
    b = jax.lax.dot_general(x, w3_ref[0], dn, preferred_element_type=jnp.float32)
    h = (a * jax.nn.sigmoid(a)) * b
    o_ref[...] = jax.lax.dot_general(h, w2_ref[0], dn,
                                     preferred_element_type=jnp.float32)


def _ffn_call(xdisp, w1, w3, w2):
    return pl.pallas_call(
        _ffn_body,
        grid=(E,),
        in_specs=[
            pl.BlockSpec((CAP, D), lambda e: (e, 0)),
            pl.BlockSpec((1, HID, D), lambda e: (e, 0, 0)),
            pl.BlockSpec((1, HID, D), lambda e: (e, 0, 0)),
            pl.BlockSpec((1, D, HID), lambda e: (e, 0, 0)),
        ],
        out_specs=pl.BlockSpec((CAP, D), lambda e: (e, 0)),
        out_shape=jax.ShapeDtypeStruct((ECAP, D), jnp.float32),
    )(xdisp, w1, w3, w2)


def _comb_body(x1_ref, y1_ref, y2_ref, rg_ref, s1_ref, s2_ref, lnf_ref, o_ref):
    k1 = s1_ref[...] < ECAP
    k2 = s2_ref[...] < ECAP
    g1 = rg_ref[..., 0:1]
    g2 = rg_ref[..., 1:2]
    moe = (jnp.where(k1, y1_ref[...], 0.0) * jnp.where(k1, g1, 0.0)
           + jnp.where(k2, y2_ref[...], 0.0) * jnp.where(k2, g2, 0.0))
    x2 = x1_ref[...] + moe
    o_ref[...] = _rms_rows(x2, lnf_ref[...])


def _comb_call(x1, y1, y2, rg, s1, s2, lnf):
    row = lambda i: (i, 0)
    full = lambda i: (0, 0)
    return pl.pallas_call(
        _comb_body,
        grid=(T // BT,),
        in_specs=[
            pl.BlockSpec((BT, D), row),
            pl.BlockSpec((BT, D), row),
            pl.BlockSpec((BT, D), row),
            pl.BlockSpec((BT, E), row),
            pl.BlockSpec((BT, 1), row),
            pl.BlockSpec((BT, 1), row),
            pl.BlockSpec((1, D), full),
        ],
        out_specs=pl.BlockSpec((BT, D), row),
        out_shape=jax.ShapeDtypeStruct((T, D), jnp.float32),
    )(x1, y1, y2, rg, s1, s2, lnf)


def _lm_body(a_ref, w_ref, o_ref):
    o_ref[...] = jnp.dot(a_ref[...], w_ref[...], preferred_element_type=jnp.float32)


def _lm_call(xf, wlmT):
    return pl.pallas_call(
        _lm_body,
        grid=(V // VB,),
        in_specs=[
            pl.BlockSpec((T, D), lambda vb: (0, 0)),
            pl.BlockSpec((D, VB), lambda vb: (0, vb)),
        ],
        out_specs=pl.BlockSpec((T, VB), lambda vb: (0, vb)),
        out_shape=jax.ShapeDtypeStruct((T, V), jnp.float32),
    )(xf, wlmT)


NC = 2
NS = 16
NW = NC * NS
TPW = T // NW

_SC_MESH = dict(core_axis_name="c", subcore_axis_name="s")


def _embed_gather(tok_emb, idxf):
    mesh = plsc.VectorSubcoreMesh(**_SC_MESH)

    @functools.partial(
        pl.kernel, mesh=mesh,
        out_type=jax.ShapeDtypeStruct((T, D), jnp.float32),
        scratch_types=[
            pltpu.VMEM((TPW,), jnp.int32),
            pltpu.VMEM((TPW, D), jnp.float32),
            pltpu.SemaphoreType.DMA,
        ],
    )
    def k(table_hbm, idx_hbm, out_hbm, idx_v, rows_v, sem):
        wid = lax.axis_index("s") * NC + lax.axis_index("c")
        base = wid * TPW
        pltpu.sync_copy(idx_hbm.at[pl.ds(base, TPW)], idx_v)
        pltpu.async_copy(table_hbm.at[idx_v], rows_v, sem).wait()
        pltpu.sync_copy(rows_v, out_hbm.at[pl.ds(base, TPW)])

    return k(tok_emb, idxf)


def _comb_gather(eo, slot1, slot2):
    mesh = plsc.VectorSubcoreMesh(**_SC_MESH)
    oy = jax.ShapeDtypeStruct((T, D), jnp.float32)

    @functools.partial(
        pl.kernel, mesh=mesh,
        out_type=[oy, oy],
        scratch_types=[
            pltpu.VMEM((TPW,), jnp.int32),
            pltpu.VMEM((TPW,), jnp.int32),
            pltpu.VMEM((TPW, D), jnp.float32),
            pltpu.VMEM((TPW, D), jnp.float32),
            pltpu.SemaphoreType.DMA,
            pltpu.SemaphoreType.DMA,
        ],
    )
    def k(eo_hbm, s1_hbm, s2_hbm, y1_hbm, y2_hbm, i1_v, i2_v, r1_v, r2_v,
          sem1, sem2):
        wid = lax.axis_index("s") * NC + lax.axis_index("c")
        base = wid * TPW
        pltpu.sync_copy(s1_hbm.at[pl.ds(base, TPW)], i1_v)
        pltpu.sync_copy(s2_hbm.at[pl.ds(base, TPW)], i2_v)
        for i in range(TPW // 16):
            sl = pl.ds(i * 16, 16)
            i1_v[sl] = jnp.minimum(i1_v[sl], ECAP - 1)
            i2_v[sl] = jnp.minimum(i2_v[sl], ECAP - 1)
        c1 = pltpu.async_copy(eo_hbm.at[i1_v], r1_v, sem1)
        c2 = pltpu.async_copy(eo_hbm.at[i2_v], r2_v, sem2)
        c1.wait()
        c2.wait()
        pltpu.sync_copy(r1_v, y1_hbm.at[pl.ds(base, TPW)])
        pltpu.sync_copy(r2_v, y2_hbm.at[pl.ds(base, TPW)])

    return k(eo, slot1, slot2)


DISP_ROWS = ECAP + NW


def _disp_scatter(slot1, slot2, x1n):
    mesh = plsc.VectorSubcoreMesh(**_SC_MESH)

    @functools.partial(
        pl.kernel, mesh=mesh,
        out_type=jax.ShapeDtypeStruct((DISP_ROWS, D), jnp.float32),
        scratch_types=[
            pltpu.VMEM((TPW,), jnp.int32),
            pltpu.VMEM((TPW,), jnp.int32),
            pltpu.VMEM((TPW, D), jnp.float32),
            pltpu.SemaphoreType.DMA,
        ],
    )
    def k(s1_hbm, s2_hbm, x_hbm, disp_hbm, d1_v, d2_v, rows_v, sem):
        cid = lax.axis_index("c")
        sid = lax.axis_index("s")
        wid = sid * NC + cid
        base = wid * TPW
        trash = ECAP + wid
        pltpu.sync_copy(s1_hbm.at[pl.ds(base, TPW)], d1_v)
        pltpu.sync_copy(s2_hbm.at[pl.ds(base, TPW)], d2_v)
        for j in range(TPW // 16):
            sl = pl.ds(j * 16, 16)
            d1_v[sl] = jnp.where(d1_v[sl] >= ECAP, trash, d1_v[sl])
            d2_v[sl] = jnp.where(d2_v[sl] >= ECAP, trash, d2_v[sl])
        pltpu.sync_copy(x_hbm.at[pl.ds(base, TPW)], rows_v)
        pltpu.async_copy(rows_v, disp_hbm.at[d1_v], sem).wait()
        pltpu.async_copy(rows_v, disp_hbm.at[d2_v], sem).wait()

    return k(slot1, slot2, x1n)


def kernel(idx, tok_emb, Wq, bq, Wk, bk, Wv, bv, Wo, bo, ln1, ln2, Wr, br, Wn, bn, w1, w3, w2, lnf, Wlm):
    x = _embed_gather(tok_emb, idx.reshape(T)).reshape(B, T, D)
    x1 = _attn_prefix(x, Wq, bq, Wk, bk, Wv, bv, Wo, bo, ln1)

    x1n = _rms_rows(x1, ln2)
    fx = x1n.reshape(T, D)
    logits = fx @ Wr.T + br
    nlog = fx @ Wn.T + bn
    eps = jax.random.normal(jax.random.key(42), (B, T, E), dtype=logits.dtype)
    noisy = logits + eps.reshape(T, E) * jax.nn.softplus(nlog)

    ri, rg, slot1, slot2 = _sel_call(noisy)
    s1f = slot1.reshape(T)
    s2f = slot2.reshape(T)
    xdisp = _disp_scatter(s1f, s2f, fx)

    eo = _ffn_call(xdisp, w1, w3, w2)

    y1, y2 = _comb_gather(eo, s1f, s2f)
    xf = _comb_call(x1.reshape(T, D), y1, y2, rg, slot1, slot2,
                    lnf.reshape(1, D))
    out = _lm_call(xf, Wlm.T)
    return out.reshape(B, T, V)


# speedup vs baseline: 1.0191x; 1.0177x over previous
import functools

import jax
import jax.numpy as jnp
from jax import lax
from jax.experimental import pallas as pl
from jax.experimental.pallas import tpu as pltpu
from jax.experimental.pallas import tpu_sc as plsc

B = 1; T = 2048; D = 768; H = 12; HD = 64; E = 8; K = 2; V = 8192
HID = 1536; BLK = 2048; EPS = 1e-5
CAP = (B * T * K) // E
ECAP = E * CAP
HHD = H * HD
BT = 256
HC = 512
VB = 1024


def _rms_rows(x, w):
    return x * jax.lax.rsqrt(jnp.mean(x * x, axis=-1, keepdims=True) + EPS) * w


def _attn_prefix(x, Wq, bq, Wk, bk, Wv, bv, Wo, bo, ln1):
    xn = _rms_rows(x, ln1)
    q = jnp.einsum('btc,hdc->bhtd', xn, Wq) + bq[None, :, None, :]
    k = jnp.einsum('btc,hdc->bhtd', xn, Wk) + bk[None, :, None, :]
    v = jnp.einsum('btc,hdc->bhtd', xn, Wv) + bv[None, :, None, :]
    inv_freq = 1.0 / (10000.0 ** (jnp.arange(0, HD, 2, dtype=jnp.float32) / HD))
    pos = jnp.arange(BLK, dtype=jnp.float32)
    sinus = pos[:, None] * inv_freq[None, :]
    cos = jnp.cos(sinus)[:T][None, None, :, :]
    sin = jnp.sin(sinus)[:T][None, None, :, :]
    q1, q2 = q[..., 0::2], q[..., 1::2]
    k1, k2 = k[..., 0::2], k[..., 1::2]
    q = jnp.concatenate([q1 * cos - q2 * sin, q1 * sin + q2 * cos], axis=-1)
    k = jnp.concatenate([k1 * cos - k2 * sin, k1 * sin + k2 * cos], axis=-1)
    scores = jnp.einsum('bhtd,bhsd->bhts', q, k) / (HD ** 0.5)
    mask = jnp.tril(jnp.ones((T, T), dtype=bool))
    scores = jnp.where(mask[None, None, :, :], scores, -jnp.inf)
    att = jax.nn.softmax(scores, axis=-1)
    out = jnp.einsum('bhts,bhsd->bhtd', att, v)
    out = out.transpose(0, 2, 1, 3).reshape(B, T, H * HD)
    return x + (out @ Wo.T + bo)


def _top2(noisy, n, lane):
    v1 = jnp.max(noisy, axis=-1, keepdims=True)
    i1 = jnp.min(jnp.where(noisy == v1, lane, E), axis=-1, keepdims=True)
    masked = jnp.where(lane == i1, -jnp.inf, noisy)
    v2 = jnp.max(masked, axis=-1, keepdims=True)
    i2 = jnp.min(jnp.where(masked == v2, lane, E), axis=-1, keepdims=True)
    return i1, i2, v1, v2


def _sel_body(noisy_ref, ri_ref, rg_ref, sel_ref):
    lane = jax.lax.broadcasted_iota(jnp.int32, (BT, E), 1)
    i1, i2, v1, v2 = _top2(noisy_ref[...], BT, lane)
    z = jnp.exp(v2 - v1)
    g1 = 1.0 / (1.0 + z)
    g2 = z / (1.0 + z)
    ri_ref[...] = jnp.where(lane == 0, i1, jnp.where(lane == 1, i2, 0))
    rg_ref[...] = jnp.where(lane == 0, g1, jnp.where(lane == 1, g2, 0.0))
    sel_ref[...] = ((lane == i1) | (lane == i2)).astype(jnp.float32)


def _sel_call(noisy):
    row = lambda i: (i, 0)
    return pl.pallas_call(
        _sel_body,
        grid=(T // BT,),
        in_specs=[pl.BlockSpec((BT, E), row)],
        out_specs=[pl.BlockSpec((BT, E), row), pl.BlockSpec((BT, E), row),
                   pl.BlockSpec((BT, E), row)],
        out_shape=[jax.ShapeDtypeStruct((T, E), jnp.int32),
                   jax.ShapeDtypeStruct((T, E), jnp.float32),
                   jax.ShapeDtypeStruct((T, E), jnp.float32)],
    )(noisy)


def _pos_body(sel_ref, ri_ref, s1_ref, s2_ref):
    rb = pl.program_id(0)
    sel = sel_ref[...]
    rows = rb * BT + jax.lax.broadcasted_iota(jnp.int32, (BT, T), 0)
    cols = jax.lax.broadcasted_iota(jnp.int32, (BT, T), 1)
    tl = (cols < rows).astype(jnp.float32)
    pos = jnp.dot(tl, sel, preferred_element_type=jnp.float32).astype(jnp.int32)
    lane = jax.lax.broadcasted_iota(jnp.int32, (BT, E), 1)
    i1 = ri_ref[..., 0:1]
    i2 = ri_ref[..., 1:2]
    p1 = jnp.sum(jnp.where(lane == i1, pos, 0), axis=-1, keepdims=True)
    p2 = jnp.sum(jnp.where(lane == i2, pos, 0), axis=-1, keepdims=True)
    s1_ref[...] = jnp.where(p1 < CAP, i1 * CAP + p1, ECAP)
    s2_ref[...] = jnp.where(p2 < CAP, i2 * CAP + p2, ECAP)


def _pos_call(sel, ri):
    row = lambda i: (i, 0)
    full = lambda i: (0, 0)
    return pl.pallas_call(
        _pos_body,
        grid=(T // BT,),
        in_specs=[pl.BlockSpec((T, E), full), pl.BlockSpec((BT, E), row)],
        out_specs=[pl.BlockSpec((BT, 1), row), pl.BlockSpec((BT, 1), row)],
        out_shape=[jax.ShapeDtypeStruct((T, 1), jnp.int32),
                   jax.ShapeDtypeStruct((T, 1), jnp.int32)],
    )(sel, ri)


def _ffn_body(x_ref, w1_ref, w3_ref, w2_ref, o_ref):
    x = x_ref[...]
    dn = (((1,), (1,)), ((), ()))
    a = jax.lax.dot_general(x, w1_ref[0], dn, preferred_element_type=jnp.float32)
    b = jax.lax.dot_general(x, w3_ref[0], dn, preferred_element_type=jnp.float32)
    h = (a * jax.nn.sigmoid(a)) * b
    o_ref[...] = jax.lax.dot_general(h, w2_ref[0], dn,
                                     preferred_element_type=jnp.float32)


def _ffn_call(xdisp, w1, w3, w2):
    return pl.pallas_call(
        _ffn_body,
        grid=(E,),
        in_specs=[
            pl.BlockSpec((CAP, D), lambda e: (e, 0)),
            pl.BlockSpec((1, HID, D), lambda e: (e, 0, 0)),
            pl.BlockSpec((1, HID, D), lambda e: (e, 0, 0)),
            pl.BlockSpec((1, D, HID), lambda e: (e, 0, 0)),
        ],
        out_specs=pl.BlockSpec((CAP, D), lambda e: (e, 0)),
        out_shape=jax.ShapeDtypeStruct((ECAP, D), jnp.float32),
    )(xdisp, w1, w3, w2)


def _comb_body(x1_ref, y1_ref, y2_ref, rg_ref, s1_ref, s2_ref, lnf_ref, o_ref):
    k1 = s1_ref[...] < ECAP
    k2 = s2_ref[...] < ECAP
    g1 = rg_ref[..., 0:1]
    g2 = rg_ref[..., 1:2]
    moe = (jnp.where(k1, y1_ref[...], 0.0) * jnp.where(k1, g1, 0.0)
           + jnp.where(k2, y2_ref[...], 0.0) * jnp.where(k2, g2, 0.0))
    x2 = x1_ref[...] + moe
    o_ref[...] = _rms_rows(x2, lnf_ref[...])


def _comb_call(x1, y1, y2, rg, s1, s2, lnf):
    row = lambda i: (i, 0)
    full = lambda i: (0, 0)
    return pl.pallas_call(
        _comb_body,
        grid=(T // BT,),
        in_specs=[
            pl.BlockSpec((BT, D), row),
            pl.BlockSpec((BT, D), row),
            pl.BlockSpec((BT, D), row),
            pl.BlockSpec((BT, E), row),
            pl.BlockSpec((BT, 1), row),
            pl.BlockSpec((BT, 1), row),
            pl.BlockSpec((1, D), full),
        ],
        out_specs=pl.BlockSpec((BT, D), row),
        out_shape=jax.ShapeDtypeStruct((T, D), jnp.float32),
    )(x1, y1, y2, rg, s1, s2, lnf)


def _lm_body(a_ref, w_ref, o_ref):
    o_ref[...] = jnp.dot(a_ref[...], w_ref[...], preferred_element_type=jnp.float32)


def _lm_call(xf, wlmT):
    return pl.pallas_call(
        _lm_body,
        grid=(V // VB,),
        in_specs=[
            pl.BlockSpec((T, D), lambda vb: (0, 0)),
            pl.BlockSpec((D, VB), lambda vb: (0, vb)),
        ],
        out_specs=pl.BlockSpec((T, VB), lambda vb: (0, vb)),
        out_shape=jax.ShapeDtypeStruct((T, V), jnp.float32),
    )(xf, wlmT)


NC = 2
NS = 16
NW = NC * NS
TPW = T // NW

_SC_MESH = dict(core_axis_name="c", subcore_axis_name="s")


def _embed_gather(tok_emb, idxf):
    mesh = plsc.VectorSubcoreMesh(**_SC_MESH)

    @functools.partial(
        pl.kernel, mesh=mesh,
        out_type=jax.ShapeDtypeStruct((T, D), jnp.float32),
        scratch_types=[
            pltpu.VMEM((TPW,), jnp.int32),
            pltpu.VMEM((TPW, D), jnp.float32),
            pltpu.SemaphoreType.DMA,
        ],
    )
    def k(table_hbm, idx_hbm, out_hbm, idx_v, rows_v, sem):
        wid = lax.axis_index("s") * NC + lax.axis_index("c")
        base = wid * TPW
        pltpu.sync_copy(idx_hbm.at[pl.ds(base, TPW)], idx_v)
        pltpu.async_copy(table_hbm.at[idx_v], rows_v, sem).wait()
        pltpu.sync_copy(rows_v, out_hbm.at[pl.ds(base, TPW)])

    return k(tok_emb, idxf)


def _comb_gather(eo, slot1, slot2):
    mesh = plsc.VectorSubcoreMesh(**_SC_MESH)
    oy = jax.ShapeDtypeStruct((T, D), jnp.float32)

    @functools.partial(
        pl.kernel, mesh=mesh,
        out_type=[oy, oy],
        scratch_types=[
            pltpu.VMEM((TPW,), jnp.int32),
            pltpu.VMEM((TPW,), jnp.int32),
            pltpu.VMEM((TPW, D), jnp.float32),
            pltpu.VMEM((TPW, D), jnp.float32),
            pltpu.SemaphoreType.DMA,
            pltpu.SemaphoreType.DMA,
        ],
    )
    def k(eo_hbm, s1_hbm, s2_hbm, y1_hbm, y2_hbm, i1_v, i2_v, r1_v, r2_v,
          sem1, sem2):
        wid = lax.axis_index("s") * NC + lax.axis_index("c")
        base = wid * TPW
        pltpu.sync_copy(s1_hbm.at[pl.ds(base, TPW)], i1_v)
        pltpu.sync_copy(s2_hbm.at[pl.ds(base, TPW)], i2_v)
        for i in range(TPW // 16):
            sl = pl.ds(i * 16, 16)
            i1_v[sl] = jnp.minimum(i1_v[sl], ECAP - 1)
            i2_v[sl] = jnp.minimum(i2_v[sl], ECAP - 1)
        c1 = pltpu.async_copy(eo_hbm.at[i1_v], r1_v, sem1)
        c2 = pltpu.async_copy(eo_hbm.at[i2_v], r2_v, sem2)
        c1.wait()
        c2.wait()
        pltpu.sync_copy(r1_v, y1_hbm.at[pl.ds(base, TPW)])
        pltpu.sync_copy(r2_v, y2_hbm.at[pl.ds(base, TPW)])

    return k(eo, slot1, slot2)


DISP_ROWS = ECAP + NW


def _disp_scatter(slot1, slot2, x1n):
    mesh = plsc.VectorSubcoreMesh(**_SC_MESH)

    @functools.partial(
        pl.kernel, mesh=mesh,
        out_type=jax.ShapeDtypeStruct((DISP_ROWS, D), jnp.float32),
        scratch_types=[
            pltpu.VMEM((TPW,), jnp.int32),
            pltpu.VMEM((TPW,), jnp.int32),
            pltpu.VMEM((TPW, D), jnp.float32),
            pltpu.SemaphoreType.DMA,
        ],
    )
    def k(s1_hbm, s2_hbm, x_hbm, disp_hbm, d1_v, d2_v, rows_v, sem):
        cid = lax.axis_index("c")
        sid = lax.axis_index("s")
        wid = sid * NC + cid
        base = wid * TPW
        trash = ECAP + wid
        pltpu.sync_copy(s1_hbm.at[pl.ds(base, TPW)], d1_v)
        pltpu.sync_copy(s2_hbm.at[pl.ds(base, TPW)], d2_v)
        for j in range(TPW // 16):
            sl = pl.ds(j * 16, 16)
            d1_v[sl] = jnp.where(d1_v[sl] >= ECAP, trash, d1_v[sl])
            d2_v[sl] = jnp.where(d2_v[sl] >= ECAP, trash, d2_v[sl])
        pltpu.sync_copy(x_hbm.at[pl.ds(base, TPW)], rows_v)
        pltpu.async_copy(rows_v, disp_hbm.at[d1_v], sem).wait()
        pltpu.async_copy(rows_v, disp_hbm.at[d2_v], sem).wait()

    return k(slot1, slot2, x1n)


def kernel(idx, tok_emb, Wq, bq, Wk, bk, Wv, bv, Wo, bo, ln1, ln2, Wr, br, Wn, bn, w1, w3, w2, lnf, Wlm):
    x = _embed_gather(tok_emb, idx.reshape(T)).reshape(B, T, D)
    x1 = _attn_prefix(x, Wq, bq, Wk, bk, Wv, bv, Wo, bo, ln1)

    x1n = _rms_rows(x1, ln2)
    fx = x1n.reshape(T, D)
    logits = fx @ Wr.T + br
    nlog = fx @ Wn.T + bn
    eps = jax.random.normal(jax.random.key(42), (B, T, E), dtype=logits.dtype)
    noisy = logits + eps.reshape(T, E) * jax.nn.softplus(nlog)

    ri, rg, sel = _sel_call(noisy)
    slot1, slot2 = _pos_call(sel, ri)
    s1f = slot1.reshape(T)
    s2f = slot2.reshape(T)
    xdisp = _disp_scatter(s1f, s2f, fx)

    eo = _ffn_call(xdisp, w1, w3, w2)

    y1, y2 = _comb_gather(eo, s1f, s2f)
    xf = _comb_call(x1.reshape(T, D), y1, y2, rg, slot1, slot2,
                    lnf.reshape(1, D))
    out = _lm_call(xf, Wlm.T)
    return out.reshape(B, T, V)
